# baseline (speedup 1.0000x reference)
import functools

import jax
import jax.numpy as jnp
from jax import lax
from jax.experimental import pallas as pl
from jax.experimental.pallas import tpu as pltpu
from jax.experimental.pallas import tpu_sc as plsc

_N = 10000
_H = 256
_HH = 128
_E = 160000
_NC = 2
_NS = 16
_KB = 128
_EPT = 10240
_EP = _EPT * _NS
_NBK = _EPT // _KB
_NG = _NBK // 8
_NSG = _NG // 2
_NPAD = 10112
_TRASH = _NPAD - 8
_RPT = _NPAD // _NS
_NBD = _EP // _KB // (_NC * _NS)


def _sc_body_seg(tab, pack, dstp, zeros128, s_out,
                 acc, pk0, pk1, dt0, dt1, rows, gsem0, gsem1):
    c = lax.axis_index("c")
    s = lax.axis_index("s")
    r0 = s * _RPT

    pltpu.sync_copy(zeros128.at[pl.ds(r0, _RPT)], acc.at[pl.ds(r0, _RPT)])

    pks = (pk0, pk1)
    dts = (dt0, dt1)
    gsems = (gsem0, gsem1)

    def load_group(gp, g):
        pltpu.sync_copy(pack.at[c, s, g], pks[gp])
        pltpu.sync_copy(dstp.at[s, g], dts[gp])

    load_group(0, 0)
    load_group(1, 1)
    plsc.subcore_barrier()

    def fire(q):
        b = q % 2
        pltpu.async_copy(tab.at[pks[(q // 8) % 2].at[q % 8]], rows.at[b],
                         gsems[b])

    def wait_and_scatter(q):
        b = q % 2
        gp = (q // 8) % 2
        pltpu.make_async_copy(tab.at[pks[gp].at[q % 8]], rows.at[b],
                              gsems[b]).wait()
        pltpu.sync_copy(rows.at[b], acc.at[dts[gp].at[q % 8]], add=True)

    fire(0)
    fire(1)

    def sgbody(sg, carry):
        for q in range(16):
            wait_and_scatter(q)
            if q == 8:
                load_group(0, 2 * sg + 2)
            elif q == 15:
                load_group(1, 2 * sg + 3)
            fire(q + 2)
        return carry

    lax.fori_loop(0, _NSG - 1, sgbody, 0)
    for q in range(16):
        wait_and_scatter(q)
        if q < 14:
            fire(q + 2)

    plsc.subcore_barrier()

    pltpu.sync_copy(acc.at[pl.ds(r0, _RPT)], s_out.at[c, pl.ds(r0, _RPT)])


def _sc_body_deg(dstp, zeros128, ones128, deg_out,
                 acc, dstall, ones_v):
    c = lax.axis_index("c")
    s = lax.axis_index("s")
    r0 = s * _RPT

    pltpu.sync_copy(zeros128.at[pl.ds(r0, _RPT)], acc.at[pl.ds(r0, _RPT)])
    pltpu.sync_copy(ones128, ones_v)
    pltpu.sync_copy(dstp.at[c * _NS + s], dstall)
    plsc.subcore_barrier()

    def batch(i, carry):
        pltpu.sync_copy(ones_v, acc.at[dstall.at[i]], add=True)
        return carry

    lax.fori_loop(0, _NBD, batch, 0)
    plsc.subcore_barrier()

    pltpu.sync_copy(acc.at[pl.ds(r0, _RPT)], deg_out.at[c, pl.ds(r0, _RPT)])


@functools.lru_cache(maxsize=None)
def _sc_kernels():
    mesh = plsc.VectorSubcoreMesh(core_axis_name="c", subcore_axis_name="s",
                                  num_cores=_NC, num_subcores=_NS)
    seg = pl.kernel(
        _sc_body_seg,
        out_type=(jax.ShapeDtypeStruct((_NC, _NPAD, _HH), jnp.float32),),
        mesh=mesh,
        scratch_types=(
            pltpu.VMEM_SHARED((_NPAD, _HH), jnp.float32),
            pltpu.VMEM((8, _KB), jnp.int32),
            pltpu.VMEM((8, _KB), jnp.int32),
            pltpu.VMEM((8, _KB), jnp.int32),
            pltpu.VMEM((8, _KB), jnp.int32),
            pltpu.VMEM((2, _KB, _HH), jnp.float32),
            pltpu.SemaphoreType.DMA,
            pltpu.SemaphoreType.DMA,
        ),
    )
    deg = pl.kernel(
        _sc_body_deg,
        out_type=(jax.ShapeDtypeStruct((_NC, _NPAD, _HH), jnp.float32),),
        mesh=mesh,
        scratch_types=(
            pltpu.VMEM_SHARED((_NPAD, _HH), jnp.float32),
            pltpu.VMEM((_NBD, _KB), jnp.int32),
            pltpu.VMEM((_KB, _HH), jnp.float32),
        ),
    )
    return seg, deg


_BN = 400


def _tc_body(s_ref, deg_ref, h_ref, wt_ref, wih_ref, whh_ref, bm_ref,
             bih_ref, bhh_ref, out_ref):
    h = h_ref[...]
    deg = deg_ref[:, 0:1]
    wt = wt_ref[...]
    f32 = jnp.float32
    a = jnp.dot(s_ref[0], wt[0:_HH], preferred_element_type=f32, precision=lax.Precision.HIGHEST)
    a = a + jnp.dot(s_ref[1], wt[_HH:_H], preferred_element_type=f32, precision=lax.Precision.HIGHEST)
    hb = jnp.dot(h, wt[_H:], preferred_element_type=f32)
    a = a + deg * (hb + bm_ref[...])
    gi = jnp.dot(a, wih_ref[...], preferred_element_type=f32) + bih_ref[...]
    gh = jnp.dot(h, whh_ref[...], preferred_element_type=f32) + bhh_ref[...]
    r = jax.nn.sigmoid(gi[:, :_H] + gh[:, :_H])
    z = jax.nn.sigmoid(gi[:, _H:2 * _H] + gh[:, _H:2 * _H])
    n = jnp.tanh(gi[:, 2 * _H:] + r * gh[:, 2 * _H:])
    out_ref[...] = (1.0 - z) * n + z * h


def _tc_round(S, deg16, h, wt, wih, whh, bm, bih, bhh):
    full = lambda shape: pl.BlockSpec(shape, lambda i: (0,) * len(shape))
    return pl.pallas_call(
        _tc_body,
        grid=(_N // _BN,),
        in_specs=[
            pl.BlockSpec((_NC, _BN, _HH), lambda i: (0, i, 0)),
            pl.BlockSpec((_BN, 16), lambda i: (i, 0)),
            pl.BlockSpec((_BN, _H), lambda i: (i, 0)),
            full((2 * _H, 2 * _H)),
            full((2 * _H, 3 * _H)),
            full((_H, 3 * _H)),
            full((1, 2 * _H)),
            full((1, 3 * _H)),
            full((1, 3 * _H)),
        ],
        out_specs=pl.BlockSpec((_BN, _H), lambda i: (i, 0)),
        out_shape=jax.ShapeDtypeStruct((_N, _H), jnp.float32),
    )(S, deg16, h, wt, wih, whh, bm, bih, bhh)


def kernel(x, edge_index, W_m1, b_m1, Wih1, Whh1, bih1, bhh1,
           W_m2, b_m2, Wih2, Whh2, bih2, bhh2):
    src = edge_index[0]
    dst = edge_index[1]
    zeros128 = jnp.zeros((_NPAD, _HH), jnp.float32)
    ones128 = jnp.ones((_KB, _HH), jnp.float32)

    pad_e = _EPT - _E // _NS
    spad = jnp.pad(src.reshape(_NS, _E // _NS), ((0, 0), (0, pad_e)))
    dpad = jnp.pad(dst.reshape(_NS, _E // _NS), ((0, 0), (0, pad_e)),
                   constant_values=_TRASH)
    spad = spad.reshape(_NS, _NG, 8, _KB)
    pack = jnp.stack([2 * spad, 2 * spad + 1], axis=0)
    dstp_seg = dpad.reshape(_NS, _NG, 8, _KB)
    dstp_deg = dpad.reshape(_NC * _NS, _NBD, _KB)

    bf = lambda v: v.astype(jnp.bfloat16).astype(jnp.float32)
    sc_seg, sc_deg = _sc_kernels()
    degF, = sc_deg(dstp_deg, zeros128, ones128)
    deg16 = (degF[0] + degF[1])[:, :16]
    S1, = sc_seg(bf(x).reshape(2 * _N, _HH), pack, dstp_seg, zeros128)
    h1 = _tc_round(S1, deg16, x, bf(W_m1.T), Wih1.T, Whh1.T,
                   b_m1[None], bih1[None], bhh1[None])
    S2, = sc_seg(bf(h1).reshape(2 * _N, _HH), pack, dstp_seg, zeros128)
    h2 = _tc_round(S2, deg16, h1, bf(W_m2.T), Wih2.T, Whh2.T,
                   b_m2[None], bih2[None], bhh2[None])
    return h2


# speedup vs baseline: 1.4004x; 1.4004x over previous
import functools

import jax
import jax.numpy as jnp
from jax import lax
from jax.experimental import pallas as pl
from jax.experimental.pallas import tpu as pltpu
from jax.experimental.pallas import tpu_sc as plsc

_N = 10000
_H = 256
_HH = 128
_E = 160000
_NC = 2
_NS = 16
_KB = 128
_EPT = _E // _NS
_NBK = _EPT // _KB
_KT = _EPT - _NBK * _KB
_NPAIR = _NBK // 2
_NPAD = 10112
_TRASH = _NPAD - 8
_RPT = _NPAD // _NS
_EPP = 10240
_NBD = _EPP * _NS // _KB // (_NC * _NS)


def _sc_body_seg(tab, src, dst, zeros128, s_out,
                 acc, idxs, idxd, gidx, rows, t_src, t_dst, t_gidx, t_rows,
                 gsem0, gsem1):
    c = lax.axis_index("c")
    s = lax.axis_index("s")
    r0 = s * _RPT

    pltpu.sync_copy(zeros128.at[pl.ds(r0, _RPT)], acc.at[pl.ds(r0, _RPT)])
    plsc.subcore_barrier()

    ebase = s * _EPT
    gsems = (gsem0, gsem1)

    def load_and_fire(b, i):
        off = ebase + i * _KB
        pltpu.sync_copy(src.at[pl.ds(off, _KB)], idxs.at[b])
        pltpu.sync_copy(dst.at[pl.ds(off, _KB)], idxd.at[b])
        for j in range(_KB // 16):
            sl = pl.ds(j * 16, 16)
            gidx[b, sl] = idxs[b, sl] * 2 + c
        pltpu.async_copy(tab.at[gidx.at[b]], rows.at[b], gsems[b])

    def wait_and_scatter(b):
        pltpu.make_async_copy(tab.at[gidx.at[b]], rows.at[b], gsems[b]).wait()
        pltpu.sync_copy(rows.at[b], acc.at[idxd.at[b]], add=True)

    load_and_fire(0, 0)
    load_and_fire(1, 1)

    def pairbody(g, carry):
        wait_and_scatter(0)
        load_and_fire(0, 2 * g + 2)
        wait_and_scatter(1)
        load_and_fire(1, 2 * g + 3)
        return carry

    lax.fori_loop(0, _NPAIR - 1, pairbody, 0)
    wait_and_scatter(0)
    wait_and_scatter(1)

    offt = ebase + _NBK * _KB
    pltpu.sync_copy(src.at[pl.ds(offt, _KT)], t_src)
    pltpu.sync_copy(dst.at[pl.ds(offt, _KT)], t_dst)
    t_gidx[...] = t_src[...] * 2 + c
    pltpu.async_copy(tab.at[t_gidx], t_rows, gsem0).wait()
    pltpu.sync_copy(t_rows, acc.at[t_dst], add=True)

    plsc.subcore_barrier()

    pltpu.sync_copy(acc.at[pl.ds(r0, _RPT)], s_out.at[c, pl.ds(r0, _RPT)])


def _sc_body_deg(dstp, zeros128, ones128, deg_out,
                 acc, dstall, ones_v, dsem):
    c = lax.axis_index("c")
    s = lax.axis_index("s")
    r0 = s * _RPT

    pltpu.sync_copy(zeros128.at[pl.ds(r0, _RPT)], acc.at[pl.ds(r0, _RPT)])
    pltpu.sync_copy(ones128, ones_v)
    pltpu.sync_copy(dstp.at[c * _NS + s], dstall)
    plsc.subcore_barrier()

    def batch(i, carry):
        pltpu.async_copy(ones_v, acc.at[dstall.at[i]], dsem, add=True)
        return carry

    lax.fori_loop(0, _NBD, batch, 0)

    def drain(i, carry):
        pltpu.make_async_copy(ones_v, acc.at[dstall.at[0]], dsem).wait()
        return carry

    lax.fori_loop(0, _NBD, drain, 0)
    plsc.subcore_barrier()

    pltpu.sync_copy(acc.at[pl.ds(r0, _RPT)], deg_out.at[c, pl.ds(r0, _RPT)])


@functools.lru_cache(maxsize=None)
def _sc_kernels():
    mesh = plsc.VectorSubcoreMesh(core_axis_name="c", subcore_axis_name="s",
                                  num_cores=_NC, num_subcores=_NS)
    seg = pl.kernel(
        _sc_body_seg,
        out_type=(jax.ShapeDtypeStruct((_NC, _NPAD, _HH), jnp.float32),),
        mesh=mesh,
        scratch_types=(
            pltpu.VMEM_SHARED((_NPAD, _HH), jnp.float32),
            pltpu.VMEM((2, _KB), jnp.int32),
            pltpu.VMEM((2, _KB), jnp.int32),
            pltpu.VMEM((2, _KB), jnp.int32),
            pltpu.VMEM((2, _KB, _HH), jnp.float32),
            pltpu.VMEM((_KT,), jnp.int32),
            pltpu.VMEM((_KT,), jnp.int32),
            pltpu.VMEM((_KT,), jnp.int32),
            pltpu.VMEM((_KT, _HH), jnp.float32),
            pltpu.SemaphoreType.DMA,
            pltpu.SemaphoreType.DMA,
        ),
    )
    deg = pl.kernel(
        _sc_body_deg,
        out_type=(jax.ShapeDtypeStruct((_NC, _NPAD, _HH), jnp.float32),),
        mesh=mesh,
        scratch_types=(
            pltpu.VMEM_SHARED((_NPAD, _HH), jnp.float32),
            pltpu.VMEM((_NBD, _KB), jnp.int32),
            pltpu.VMEM((_KB, _HH), jnp.float32),
            pltpu.SemaphoreType.DMA,
        ),
    )
    return seg, deg


_BN = 400


def _tc_body(s_ref, deg_ref, h_ref, wt_ref, wih_ref, whh_ref, bm_ref,
             bih_ref, bhh_ref, out_ref):
    h = h_ref[...]
    deg = deg_ref[:, 0:1]
    wt = wt_ref[...]
    f32 = jnp.float32
    a = jnp.dot(s_ref[0], wt[0:_HH], preferred_element_type=f32,
                precision=lax.Precision.HIGHEST)
    a = a + jnp.dot(s_ref[1], wt[_HH:_H], preferred_element_type=f32,
                    precision=lax.Precision.HIGHEST)
    hb = jnp.dot(h, wt[_H:], preferred_element_type=f32)
    a = a + deg * (hb + bm_ref[...])
    gi = jnp.dot(a, wih_ref[...], preferred_element_type=f32) + bih_ref[...]
    gh = jnp.dot(h, whh_ref[...], preferred_element_type=f32) + bhh_ref[...]
    r = jax.nn.sigmoid(gi[:, :_H] + gh[:, :_H])
    z = jax.nn.sigmoid(gi[:, _H:2 * _H] + gh[:, _H:2 * _H])
    n = jnp.tanh(gi[:, 2 * _H:] + r * gh[:, 2 * _H:])
    out_ref[...] = (1.0 - z) * n + z * h


def _tc_round(S, deg16, h, wt, wih, whh, bm, bih, bhh):
    full = lambda shape: pl.BlockSpec(shape, lambda i: (0,) * len(shape))
    return pl.pallas_call(
        _tc_body,
        grid=(_N // _BN,),
        in_specs=[
            pl.BlockSpec((_NC, _BN, _HH), lambda i: (0, i, 0)),
            pl.BlockSpec((_BN, 16), lambda i: (i, 0)),
            pl.BlockSpec((_BN, _H), lambda i: (i, 0)),
            full((2 * _H, 2 * _H)),
            full((2 * _H, 3 * _H)),
            full((_H, 3 * _H)),
            full((1, 2 * _H)),
            full((1, 3 * _H)),
            full((1, 3 * _H)),
        ],
        out_specs=pl.BlockSpec((_BN, _H), lambda i: (i, 0)),
        out_shape=jax.ShapeDtypeStruct((_N, _H), jnp.float32),
    )(S, deg16, h, wt, wih, whh, bm, bih, bhh)


def kernel(x, edge_index, W_m1, b_m1, Wih1, Whh1, bih1, bhh1,
           W_m2, b_m2, Wih2, Whh2, bih2, bhh2):
    src = edge_index[0]
    dst = edge_index[1]
    zeros128 = jnp.zeros((_NPAD, _HH), jnp.float32)
    ones128 = jnp.ones((_KB, _HH), jnp.float32)

    pad_e = _EPP - _E // _NS
    dpad = jnp.pad(dst.reshape(_NS, _E // _NS), ((0, 0), (0, pad_e)),
                   constant_values=_TRASH)
    dstp_deg = dpad.reshape(_NC * _NS, _NBD, _KB)

    bf = lambda v: v.astype(jnp.bfloat16).astype(jnp.float32)
    sc_seg, sc_deg = _sc_kernels()
    degF, = sc_deg(dstp_deg, zeros128, ones128)
    deg16 = (degF[0] + degF[1])[:, :16]
    S1, = sc_seg(bf(x).reshape(2 * _N, _HH), src, dst, zeros128)
    h1 = _tc_round(S1, deg16, x, bf(W_m1.T), Wih1.T, Whh1.T,
                   b_m1[None], bih1[None], bhh1[None])
    S2, = sc_seg(bf(h1).reshape(2 * _N, _HH), src, dst, zeros128)
    h2 = _tc_round(S2, deg16, h1, bf(W_m2.T), Wih2.T, Whh2.T,
                   b_m2[None], bih2[None], bhh2[None])
    return h2
